# baseline (device time: 85600 ns/iter reference)
import jax
import jax.numpy as jnp
from jax import lax
from jax.experimental import pallas as pl
from jax.experimental.pallas import tpu as pltpu

N_DEV = 16
B, SQ, SKV = 2, 256, 256
H_LOCAL, DH = 4, 64
D_MODEL = 512
ROWS = B * SQ
COLS = D_MODEL
CH = ROWS // N_DEV


def _allreduce_body(p_ref, out_ref, comm_ref, rs_send, rs_recv, ag_send, ag_recv):
    me = lax.axis_index("i")
    left = (me - 1) % N_DEV
    right = (me + 1) % N_DEV

    barrier = pltpu.get_barrier_semaphore()
    for nbr in (left, right):
        pl.semaphore_signal(
            barrier, inc=1, device_id=(nbr,), device_id_type=pl.DeviceIdType.MESH
        )
    pl.semaphore_wait(barrier, 2)

    out_ref[...] = p_ref[...]

    for s in range(N_DEV - 1):
        send_c = (me - s) % N_DEV
        rdma = pltpu.make_async_remote_copy(
            src_ref=out_ref.at[pl.ds(send_c * CH, CH), :],
            dst_ref=comm_ref.at[s],
            send_sem=rs_send.at[s],
            recv_sem=rs_recv.at[s],
            device_id=(right,),
            device_id_type=pl.DeviceIdType.MESH,
        )
        rdma.start()
        rdma.wait()
        recv_c = (me - s - 1) % N_DEV
        sl = pl.ds(recv_c * CH, CH)
        out_ref[sl, :] = out_ref[sl, :] + comm_ref[s]

    for s in range(N_DEV - 1):
        send_c = (me + 1 - s) % N_DEV
        sl = pl.ds(send_c * CH, CH)
        rdma = pltpu.make_async_remote_copy(
            src_ref=out_ref.at[sl, :],
            dst_ref=out_ref.at[sl, :],
            send_sem=ag_send.at[s],
            recv_sem=ag_recv.at[s],
            device_id=(right,),
            device_id_type=pl.DeviceIdType.MESH,
        )
        rdma.start()
        rdma.wait()


def _ring_allreduce(partial):
    return pl.pallas_call(
        _allreduce_body,
        out_shape=jax.ShapeDtypeStruct((ROWS, COLS), jnp.float32),
        in_specs=[pl.BlockSpec(memory_space=pltpu.VMEM)],
        out_specs=pl.BlockSpec(memory_space=pltpu.VMEM),
        scratch_shapes=[
            pltpu.VMEM((N_DEV - 1, CH, COLS), jnp.float32),
            pltpu.SemaphoreType.DMA((N_DEV - 1,)),
            pltpu.SemaphoreType.DMA((N_DEV - 1,)),
            pltpu.SemaphoreType.DMA((N_DEV - 1,)),
            pltpu.SemaphoreType.DMA((N_DEV - 1,)),
        ],
        compiler_params=pltpu.CompilerParams(collective_id=0),
    )(partial)


def kernel(x, Wq, K_ext, V_ext, Wo):
    i = lax.axis_index("i")
    bf = jnp.bfloat16

    Q = jnp.einsum(
        "bsd,dh->bsh", x.astype(bf), Wq.astype(bf), preferred_element_type=jnp.float32
    ).reshape(B, SQ, H_LOCAL, DH)
    Kh = lax.dynamic_slice_in_dim(K_ext, i * H_LOCAL, H_LOCAL, axis=2)
    Vh = lax.dynamic_slice_in_dim(V_ext, i * H_LOCAL, H_LOCAL, axis=2)

    scores = (
        jnp.einsum(
            "bihd,bjhd->bhij",
            Q.astype(bf),
            Kh.astype(bf),
            preferred_element_type=jnp.float32,
        )
        * 0.125
    )
    qi = jnp.arange(SQ)[:, None]
    ki = jnp.arange(SKV)[None, :]
    mask = (jnp.abs(qi - ki) <= 128) | (ki < 32) | (qi < 32)
    scores = jnp.where(mask[None, None], scores, -1e9)
    m = scores.max(axis=-1, keepdims=True)
    w = jnp.exp(scores - m)
    w = w / w.sum(axis=-1, keepdims=True)

    ctx = jnp.einsum(
        "bhij,bjhd->bihd",
        w.astype(bf),
        Vh.astype(bf),
        preferred_element_type=jnp.float32,
    ).reshape(B, SQ, H_LOCAL * DH)

    partial = jnp.einsum(
        "bsf,fd->bsd",
        ctx.astype(bf),
        Wo.astype(bf),
        preferred_element_type=jnp.float32,
    )

    out = _ring_allreduce(partial.reshape(ROWS, COLS))
    return out.reshape(B, SQ, D_MODEL)


# device time: 46235 ns/iter; 1.8514x vs baseline; 1.8514x over previous
import jax
import jax.numpy as jnp
from jax import lax
from jax.experimental import pallas as pl
from jax.experimental.pallas import tpu as pltpu

N_DEV = 16
B, SQ, SKV = 2, 256, 256
H_LOCAL, DH = 4, 64
D_MODEL = 512
ROWS = B * SQ
COLS = D_MODEL

DIST = (1, 4, 2, 8)
HALF = (256, 128, 64, 32)
OFF = (0, 256, 384, 448)
COMM_ROWS = 480


def _allreduce_body(p_ref, out_ref, comm_ref, rs_send, rs_recv, ag_send, ag_recv):
    me = lax.axis_index("i")

    barrier = pltpu.get_barrier_semaphore()
    for d in DIST:
        pl.semaphore_signal(
            barrier, inc=1, device_id=(me ^ d,), device_id_type=pl.DeviceIdType.MESH
        )
    pl.semaphore_wait(barrier, len(DIST))

    out_ref[...] = p_ref[...]

    seg_start = me * 0
    for k, d in enumerate(DIST):
        half = HALF[k]
        mybit = (me // d) % 2
        keep_start = seg_start + mybit * half
        send_start = seg_start + (1 - mybit) * half
        rdma = pltpu.make_async_remote_copy(
            src_ref=out_ref.at[pl.ds(send_start, half), :],
            dst_ref=comm_ref.at[pl.ds(OFF[k], half), :],
            send_sem=rs_send.at[k],
            recv_sem=rs_recv.at[k],
            device_id=(me ^ d,),
            device_id_type=pl.DeviceIdType.MESH,
        )
        rdma.start()
        rdma.wait()
        sl = pl.ds(keep_start, half)
        out_ref[sl, :] = out_ref[sl, :] + comm_ref[pl.ds(OFF[k], half), :]
        seg_start = keep_start

    seg_len = ROWS // N_DEV
    for k in reversed(range(len(DIST))):
        d = DIST[k]
        sl = pl.ds(seg_start, seg_len)
        rdma = pltpu.make_async_remote_copy(
            src_ref=out_ref.at[sl, :],
            dst_ref=out_ref.at[sl, :],
            send_sem=ag_send.at[k],
            recv_sem=ag_recv.at[k],
            device_id=(me ^ d,),
            device_id_type=pl.DeviceIdType.MESH,
        )
        rdma.start()
        rdma.wait()
        mybit = (me // d) % 2
        seg_start = seg_start - mybit * seg_len
        seg_len *= 2


def _butterfly_allreduce(partial):
    n_steps = len(DIST)
    return pl.pallas_call(
        _allreduce_body,
        out_shape=jax.ShapeDtypeStruct((ROWS, COLS), jnp.float32),
        in_specs=[pl.BlockSpec(memory_space=pltpu.VMEM)],
        out_specs=pl.BlockSpec(memory_space=pltpu.VMEM),
        scratch_shapes=[
            pltpu.VMEM((COMM_ROWS, COLS), jnp.float32),
            pltpu.SemaphoreType.DMA((n_steps,)),
            pltpu.SemaphoreType.DMA((n_steps,)),
            pltpu.SemaphoreType.DMA((n_steps,)),
            pltpu.SemaphoreType.DMA((n_steps,)),
        ],
        compiler_params=pltpu.CompilerParams(collective_id=0),
    )(partial)


def kernel(x, Wq, K_ext, V_ext, Wo):
    i = lax.axis_index("i")
    bf = jnp.bfloat16

    Q = jnp.einsum(
        "bsd,dh->bsh", x.astype(bf), Wq.astype(bf), preferred_element_type=jnp.float32
    ).reshape(B, SQ, H_LOCAL, DH)
    Kh = lax.dynamic_slice_in_dim(K_ext, i * H_LOCAL, H_LOCAL, axis=2)
    Vh = lax.dynamic_slice_in_dim(V_ext, i * H_LOCAL, H_LOCAL, axis=2)

    scores = (
        jnp.einsum(
            "bihd,bjhd->bhij",
            Q.astype(bf),
            Kh.astype(bf),
            preferred_element_type=jnp.float32,
        )
        * 0.125
    )
    qi = jnp.arange(SQ)[:, None]
    ki = jnp.arange(SKV)[None, :]
    mask = (jnp.abs(qi - ki) <= 128) | (ki < 32) | (qi < 32)
    scores = jnp.where(mask[None, None], scores, -1e9)
    m = scores.max(axis=-1, keepdims=True)
    w = jnp.exp(scores - m)
    w = w / w.sum(axis=-1, keepdims=True)

    ctx = jnp.einsum(
        "bhij,bjhd->bihd",
        w.astype(bf),
        Vh.astype(bf),
        preferred_element_type=jnp.float32,
    ).reshape(B, SQ, H_LOCAL * DH)

    partial = jnp.einsum(
        "bsf,fd->bsd",
        ctx.astype(bf),
        Wo.astype(bf),
        preferred_element_type=jnp.float32,
    )

    out = _butterfly_allreduce(partial.reshape(ROWS, COLS))
    return out.reshape(B, SQ, D_MODEL)


# device time: 35618 ns/iter; 2.4033x vs baseline; 1.2981x over previous
import jax
import jax.numpy as jnp
from jax import lax
from jax.experimental import pallas as pl
from jax.experimental.pallas import tpu as pltpu

N_DEV = 16
B, SQ, SKV = 2, 256, 256
H_LOCAL, DH = 4, 64
D_MODEL = 512
ROWS = B * SQ
COLS = D_MODEL

DIST = (1, 4, 2, 8)
HALF = (256, 128, 64, 32)
OFF = (0, 256, 384, 448)
COMM_ROWS = 480


def _allreduce_body(
    p_ref, out_ref, acc_ref, stage_ref, comm_ref, rs_send, rs_recv, ag_send, ag_recv
):
    me = lax.axis_index("i")

    barrier = pltpu.get_barrier_semaphore()
    for d in DIST:
        pl.semaphore_signal(
            barrier, inc=1, device_id=(me ^ d,), device_id_type=pl.DeviceIdType.MESH
        )
    pl.semaphore_wait(barrier, len(DIST))

    acc_ref[...] = p_ref[...]

    seg_start = me * 0
    for k, d in enumerate(DIST):
        half = HALF[k]
        mybit = (me // d) % 2
        keep_start = seg_start + mybit * half
        send_start = seg_start + (1 - mybit) * half
        stage_ref[pl.ds(0, half), :] = acc_ref[pl.ds(send_start, half), :].astype(
            jnp.bfloat16
        )
        rdma = pltpu.make_async_remote_copy(
            src_ref=stage_ref.at[pl.ds(0, half), :],
            dst_ref=comm_ref.at[pl.ds(OFF[k], half), :],
            send_sem=rs_send.at[k],
            recv_sem=rs_recv.at[k],
            device_id=(me ^ d,),
            device_id_type=pl.DeviceIdType.MESH,
        )
        rdma.start()
        rdma.wait()
        sl = pl.ds(keep_start, half)
        out_sl = comm_ref[pl.ds(OFF[k], half), :].astype(jnp.float32)
        acc_ref[sl, :] = acc_ref[sl, :] + out_sl
        seg_start = keep_start

    seg_len = ROWS // N_DEV
    out_ref[pl.ds(seg_start, seg_len), :] = acc_ref[
        pl.ds(seg_start, seg_len), :
    ].astype(jnp.bfloat16)
    for k in reversed(range(len(DIST))):
        d = DIST[k]
        sl = pl.ds(seg_start, seg_len)
        rdma = pltpu.make_async_remote_copy(
            src_ref=out_ref.at[sl, :],
            dst_ref=out_ref.at[sl, :],
            send_sem=ag_send.at[k],
            recv_sem=ag_recv.at[k],
            device_id=(me ^ d,),
            device_id_type=pl.DeviceIdType.MESH,
        )
        rdma.start()
        rdma.wait()
        mybit = (me // d) % 2
        seg_start = seg_start - mybit * seg_len
        seg_len *= 2


def _butterfly_allreduce(partial):
    n_steps = len(DIST)
    return pl.pallas_call(
        _allreduce_body,
        out_shape=jax.ShapeDtypeStruct((ROWS, COLS), jnp.bfloat16),
        in_specs=[pl.BlockSpec(memory_space=pltpu.VMEM)],
        out_specs=pl.BlockSpec(memory_space=pltpu.VMEM),
        scratch_shapes=[
            pltpu.VMEM((ROWS, COLS), jnp.float32),
            pltpu.VMEM((HALF[0], COLS), jnp.bfloat16),
            pltpu.VMEM((COMM_ROWS, COLS), jnp.bfloat16),
            pltpu.SemaphoreType.DMA((n_steps,)),
            pltpu.SemaphoreType.DMA((n_steps,)),
            pltpu.SemaphoreType.DMA((n_steps,)),
            pltpu.SemaphoreType.DMA((n_steps,)),
        ],
        compiler_params=pltpu.CompilerParams(collective_id=0),
    )(partial)


def kernel(x, Wq, K_ext, V_ext, Wo):
    i = lax.axis_index("i")
    bf = jnp.bfloat16

    Q = jnp.einsum(
        "bsd,dh->bsh", x.astype(bf), Wq.astype(bf), preferred_element_type=jnp.float32
    ).reshape(B, SQ, H_LOCAL, DH)
    Kh = lax.dynamic_slice_in_dim(K_ext, i * H_LOCAL, H_LOCAL, axis=2)
    Vh = lax.dynamic_slice_in_dim(V_ext, i * H_LOCAL, H_LOCAL, axis=2)

    scores = (
        jnp.einsum(
            "bihd,bjhd->bhij",
            Q.astype(bf),
            Kh.astype(bf),
            preferred_element_type=jnp.float32,
        )
        * 0.125
    )
    qi = jnp.arange(SQ)[:, None]
    ki = jnp.arange(SKV)[None, :]
    mask = (jnp.abs(qi - ki) <= 128) | (ki < 32) | (qi < 32)
    scores = jnp.where(mask[None, None], scores, -1e9)
    m = scores.max(axis=-1, keepdims=True)
    w = jnp.exp(scores - m)
    w = w / w.sum(axis=-1, keepdims=True)

    ctx = jnp.einsum(
        "bhij,bjhd->bihd",
        w.astype(bf),
        Vh.astype(bf),
        preferred_element_type=jnp.float32,
    ).reshape(B, SQ, H_LOCAL * DH)

    partial = jnp.einsum(
        "bsf,fd->bsd",
        ctx.astype(bf),
        Wo.astype(bf),
        preferred_element_type=jnp.float32,
    )

    out = _butterfly_allreduce(partial.reshape(ROWS, COLS))
    return out.reshape(B, SQ, D_MODEL)


# device time: 25417 ns/iter; 3.3678x vs baseline; 1.4013x over previous
import jax
import jax.numpy as jnp
from jax import lax
from jax.experimental import pallas as pl
from jax.experimental.pallas import tpu as pltpu

N_DEV = 16
B, SQ, SKV = 2, 256, 256
H_LOCAL, DH = 4, 64
D_MODEL = 512
ROWS = B * SQ
COLS = D_MODEL
CH = ROWS // N_DEV


def _allreduce_body(
    p_ref, out_ref, stage_ref, comm_ref, rs_send, rs_recv, ag_send, ag_recv
):
    me = lax.axis_index("i")

    barrier = pltpu.get_barrier_semaphore()
    for off in range(1, N_DEV):
        pl.semaphore_signal(
            barrier,
            inc=1,
            device_id=((me + off) % N_DEV,),
            device_id_type=pl.DeviceIdType.MESH,
        )
    pl.semaphore_wait(barrier, N_DEV - 1)

    stage_ref[...] = p_ref[...].astype(jnp.bfloat16)

    rs_sends = []
    for off in range(1, N_DEV):
        tgt = (me + off) % N_DEV
        rdma = pltpu.make_async_remote_copy(
            src_ref=stage_ref.at[pl.ds(tgt * CH, CH), :],
            dst_ref=comm_ref.at[me],
            send_sem=rs_send.at[off],
            recv_sem=rs_recv.at[off],
            device_id=(tgt,),
            device_id_type=pl.DeviceIdType.MESH,
        )
        rdma.start()
        rs_sends.append(rdma)

    acc = p_ref[pl.ds(me * CH, CH), :]
    for off in range(1, N_DEV):
        src = (me - off) % N_DEV
        recv = pltpu.make_async_remote_copy(
            src_ref=comm_ref.at[src],
            dst_ref=comm_ref.at[src],
            send_sem=rs_send.at[off],
            recv_sem=rs_recv.at[off],
            device_id=(src,),
            device_id_type=pl.DeviceIdType.MESH,
        )
        recv.wait_recv()
        acc = acc + comm_ref[src].astype(jnp.float32)

    out_ref[pl.ds(me * CH, CH), :] = acc.astype(jnp.bfloat16)

    ag_sends = []
    for off in range(1, N_DEV):
        tgt = (me + off) % N_DEV
        rdma = pltpu.make_async_remote_copy(
            src_ref=out_ref.at[pl.ds(me * CH, CH), :],
            dst_ref=out_ref.at[pl.ds(me * CH, CH), :],
            send_sem=ag_send.at[off],
            recv_sem=ag_recv.at[off],
            device_id=(tgt,),
            device_id_type=pl.DeviceIdType.MESH,
        )
        rdma.start()
        ag_sends.append(rdma)

    for off in range(1, N_DEV):
        src = (me - off) % N_DEV
        recv = pltpu.make_async_remote_copy(
            src_ref=out_ref.at[pl.ds(src * CH, CH), :],
            dst_ref=out_ref.at[pl.ds(src * CH, CH), :],
            send_sem=ag_send.at[off],
            recv_sem=ag_recv.at[off],
            device_id=(src,),
            device_id_type=pl.DeviceIdType.MESH,
        )
        recv.wait_recv()

    for rdma in rs_sends + ag_sends:
        rdma.wait_send()


def _alltoall_allreduce(partial):
    return pl.pallas_call(
        _allreduce_body,
        out_shape=jax.ShapeDtypeStruct((ROWS, COLS), jnp.bfloat16),
        in_specs=[pl.BlockSpec(memory_space=pltpu.VMEM)],
        out_specs=pl.BlockSpec(memory_space=pltpu.VMEM),
        scratch_shapes=[
            pltpu.VMEM((ROWS, COLS), jnp.bfloat16),
            pltpu.VMEM((N_DEV, CH, COLS), jnp.bfloat16),
            pltpu.SemaphoreType.DMA((N_DEV,)),
            pltpu.SemaphoreType.DMA((N_DEV,)),
            pltpu.SemaphoreType.DMA((N_DEV,)),
            pltpu.SemaphoreType.DMA((N_DEV,)),
        ],
        compiler_params=pltpu.CompilerParams(collective_id=0),
    )(partial)


def kernel(x, Wq, K_ext, V_ext, Wo):
    i = lax.axis_index("i")
    bf = jnp.bfloat16

    Q = jnp.einsum(
        "bsd,dh->bsh", x.astype(bf), Wq.astype(bf), preferred_element_type=jnp.float32
    ).reshape(B, SQ, H_LOCAL, DH)
    Kh = lax.dynamic_slice_in_dim(K_ext, i * H_LOCAL, H_LOCAL, axis=2)
    Vh = lax.dynamic_slice_in_dim(V_ext, i * H_LOCAL, H_LOCAL, axis=2)

    scores = (
        jnp.einsum(
            "bihd,bjhd->bhij",
            Q.astype(bf),
            Kh.astype(bf),
            preferred_element_type=jnp.float32,
        )
        * 0.125
    )
    qi = jnp.arange(SQ)[:, None]
    ki = jnp.arange(SKV)[None, :]
    mask = (jnp.abs(qi - ki) <= 128) | (ki < 32) | (qi < 32)
    scores = jnp.where(mask[None, None], scores, -1e9)
    m = scores.max(axis=-1, keepdims=True)
    w = jnp.exp(scores - m)
    w = w / w.sum(axis=-1, keepdims=True)

    ctx = jnp.einsum(
        "bhij,bjhd->bihd",
        w.astype(bf),
        Vh.astype(bf),
        preferred_element_type=jnp.float32,
    ).reshape(B, SQ, H_LOCAL * DH)

    partial = jnp.einsum(
        "bsf,fd->bsd",
        ctx.astype(bf),
        Wo.astype(bf),
        preferred_element_type=jnp.float32,
    )

    out = _alltoall_allreduce(partial.reshape(ROWS, COLS))
    return out.reshape(B, SQ, D_MODEL)
